# baseline (speedup 1.0000x reference)
import jax
import jax.numpy as jnp
from jax.experimental import pallas as pl
from jax.experimental.pallas import tpu as pltpu

_T = 9.210440366976517


def _focal_terms(x, g, v):
    xc = jnp.clip(x, -_T, _T)
    e = jnp.exp(xc)
    one_m_p = 1.0 / (1.0 + e)
    p = e * one_m_p
    sp = jnp.log1p(e)
    log_p = xc - sp
    log_1mp = -sp
    posf = (g == 1.0).astype(jnp.float32)
    negf = (g < 1.0).astype(jnp.float32)
    w = 1.0 - g
    w2 = w * w
    neg_w = w2 * w2
    s = jnp.sum((log_p * one_m_p * one_m_p * posf
                 + log_1mp * p * p * neg_w * negf) * v)
    return s, jnp.sum(posf)


def _gather_hw(orow, lcol, img):
    rows = jax.lax.dot_general(orow, img, (((1,), (0,)), ((), ())),
                               preferred_element_type=jnp.float32)
    return jnp.sum(rows * lcol, axis=1)


def _make_body(nsteps, B, K, H, W):
    def body(ht, hb, gt, gb, valt, valb,
             a0, ga0, a1, ga1, a2, ga2,
             tagt, tagb, offt, offb, indt, indb, mk, got, gob,
             out, acc):
        i = pl.program_id(0)

        @pl.when(i == 0)
        def _init():
            acc[0] = 0.0
            acc[1] = 0.0
            acc[2] = 0.0
            acc[3] = 0.0

        s_tl, n_tl = _focal_terms(ht[...], gt[...], valt[...])
        s_br, n_br = _focal_terms(hb[...], gb[...], valb[...])
        acc[0] = acc[0] + s_tl
        acc[1] = acc[1] + n_tl
        acc[2] = acc[2] + s_br
        acc[3] = acc[3] + n_br

        @pl.when(i == 0)
        def _small():
            def focal(x, g):
                s, n = _focal_terms(x, g, 1.0)
                return -s / n

            att = (focal(a0[...], ga0[...])
                   + focal(a1[...], ga1[...])
                   + focal(a2[...], ga2[...]))

            mkf = mk[...]
            num_tot = jnp.sum(mkf)
            iota_h = jax.lax.broadcasted_iota(jnp.int32, (K, H), 1)
            iota_w = jax.lax.broadcasted_iota(jnp.int32, (K, W), 1)
            pull = 0.0
            osum = 0.0
            for b in range(B):
                it = indt[b, :]
                ib = indb[b, :]
                orow_t = (it[:, None] // W == iota_h).astype(jnp.float32)
                lcol_t = (it[:, None] % W == iota_w).astype(jnp.float32)
                orow_b = (ib[:, None] // W == iota_h).astype(jnp.float32)
                lcol_b = (ib[:, None] % W == iota_w).astype(jnp.float32)
                mb = mkf[b, :]
                nb = jnp.sum(mb)
                t0 = _gather_hw(orow_t, lcol_t, tagt[b])
                t1 = _gather_hw(orow_b, lcol_b, tagb[b])
                dtag = t0 - t1
                pull = pull + jnp.sum(dtag * dtag * 0.5 / (nb + 1e-4) * mb)
                for orow, lcol, off, go in (
                        (orow_t, lcol_t, offt, got),
                        (orow_b, lcol_b, offb, gob)):
                    for c in range(2):
                        d = _gather_hw(orow, lcol, off[b, c]) - go[b, c, :]
                        ad = jnp.abs(d)
                        l = jnp.where(ad < 1.0, 0.5 * d * d, ad - 0.5)
                        osum = osum + jnp.sum(l * mb)

            acc[4] = att + pull + osum / (num_tot + 1e-4)

        @pl.when(i == nsteps - 1)
        def _final():
            total = -acc[0] / acc[1] - acc[2] / acc[3] + acc[4]
            out[...] = jnp.broadcast_to(total, (1, 1))

    return body


def kernel(tl_heat, br_heat, tl_tag, br_tag, tl_off, br_off,
           att0, att1, att2, gt_tl_heat, gt_br_heat, gt_mask,
           gt_tl_off, gt_br_off, gt_tl_ind, gt_br_ind,
           gt_tl_valid, gt_br_valid, gt_att0, gt_att1, gt_att2):
    B, C, H, W = tl_heat.shape
    K = gt_mask.shape[1]
    R = B * H
    ROWS = 64
    nsteps = R // ROWS

    big = [jnp.transpose(a, (0, 2, 3, 1)).reshape(R, W, C) for a in
           (tl_heat, br_heat, gt_tl_heat, gt_br_heat,
            gt_tl_valid, gt_br_valid)]
    a0 = att0.reshape(B, *att0.shape[2:])
    ga0 = gt_att0.reshape(B, *gt_att0.shape[2:])
    a1 = att1.reshape(B, *att1.shape[2:])
    ga1 = gt_att1.reshape(B, *gt_att1.shape[2:])
    a2 = att2.reshape(B, *att2.shape[2:])
    ga2 = gt_att2.reshape(B, *gt_att2.shape[2:])
    tag_tl = tl_tag.reshape(B, H, W)
    tag_br = br_tag.reshape(B, H, W)
    ind_tl = gt_tl_ind.astype(jnp.int32)
    ind_br = gt_br_ind.astype(jnp.int32)
    maskf = gt_mask.astype(jnp.float32)
    goff_tl = jnp.transpose(gt_tl_off, (0, 2, 1))
    goff_br = jnp.transpose(gt_br_off, (0, 2, 1))

    big_spec = pl.BlockSpec((ROWS, W, C), lambda i: (i, 0, 0))
    full = lambda shape: pl.BlockSpec(shape, lambda i: (0,) * len(shape))

    res = pl.pallas_call(
        _make_body(nsteps, B, K, H, W),
        grid=(nsteps,),
        in_specs=[big_spec] * 6 + [
            full(a0.shape), full(ga0.shape),
            full(a1.shape), full(ga1.shape),
            full(a2.shape), full(ga2.shape),
            full(tag_tl.shape), full(tag_br.shape),
            full(tl_off.shape), full(br_off.shape),
            full(ind_tl.shape), full(ind_br.shape),
            full(maskf.shape),
            full(goff_tl.shape), full(goff_br.shape),
        ],
        out_specs=pl.BlockSpec((1, 1), lambda i: (0, 0)),
        out_shape=jax.ShapeDtypeStruct((1, 1), jnp.float32),
        scratch_shapes=[pltpu.SMEM((5,), jnp.float32)],
        compiler_params=pltpu.CompilerParams(
            dimension_semantics=("arbitrary",)),
    )(*big, a0, ga0, a1, ga1, a2, ga2,
      tag_tl, tag_br, tl_off, br_off, ind_tl, ind_br, maskf,
      goff_tl, goff_br)
    return res.reshape(1)


# speedup vs baseline: 1.0466x; 1.0466x over previous
import jax
import jax.numpy as jnp
from jax.experimental import pallas as pl
from jax.experimental.pallas import tpu as pltpu

_T = 9.210440366976517


def _focal_terms(x, g, v):
    xc = jnp.clip(x, -_T, _T)
    e = jnp.exp(xc)
    one_m_p = 1.0 / (1.0 + e)
    p = e * one_m_p
    sp = jnp.log1p(e)
    log_p = xc - sp
    log_1mp = -sp
    posf = (g == 1.0).astype(jnp.float32)
    negf = (g < 1.0).astype(jnp.float32)
    w = 1.0 - g
    w2 = w * w
    neg_w = w2 * w2
    s = jnp.sum((log_p * one_m_p * one_m_p * posf
                 + log_1mp * p * p * neg_w * negf) * v)
    return s, jnp.sum(posf)


def _gather_hw(orow, lcol, img):
    rows = jax.lax.dot_general(orow, img, (((1,), (0,)), ((), ())),
                               preferred_element_type=jnp.float32)
    return jnp.sum(rows * lcol, axis=1)


def _make_body(nsteps, B, K, H, W):
    def body(ht, hb, gt, gb, valt, valb,
             a0, ga0, a1, ga1, a2, ga2,
             tagt, tagb, offt, offb, indt, indb, mk, got, gob,
             out, acc):
        i = pl.program_id(0)

        @pl.when(i == 0)
        def _init():
            acc[0] = 0.0
            acc[1] = 0.0
            acc[2] = 0.0
            acc[3] = 0.0

        s_tl, n_tl = _focal_terms(ht[...], gt[...], valt[...])
        s_br, n_br = _focal_terms(hb[...], gb[...], valb[...])
        acc[0] = acc[0] + s_tl
        acc[1] = acc[1] + n_tl
        acc[2] = acc[2] + s_br
        acc[3] = acc[3] + n_br

        @pl.when(i == 0)
        def _small():
            def focal(x, g):
                s, n = _focal_terms(x, g, 1.0)
                return -s / n

            att = (focal(a0[...], ga0[...])
                   + focal(a1[...], ga1[...])
                   + focal(a2[...], ga2[...]))

            mkf = mk[...]
            num_tot = jnp.sum(mkf)
            iota_h = jax.lax.broadcasted_iota(jnp.int32, (K, H), 1)
            iota_w = jax.lax.broadcasted_iota(jnp.int32, (K, W), 1)
            pull = 0.0
            osum = 0.0
            for b in range(B):
                it = indt[b, :]
                ib = indb[b, :]
                orow_t = (it[:, None] // W == iota_h).astype(jnp.float32)
                lcol_t = (it[:, None] % W == iota_w).astype(jnp.float32)
                orow_b = (ib[:, None] // W == iota_h).astype(jnp.float32)
                lcol_b = (ib[:, None] % W == iota_w).astype(jnp.float32)
                mb = mkf[b, :]
                nb = jnp.sum(mb)
                t0 = _gather_hw(orow_t, lcol_t, tagt[b])
                t1 = _gather_hw(orow_b, lcol_b, tagb[b])
                dtag = t0 - t1
                pull = pull + jnp.sum(dtag * dtag * 0.5 / (nb + 1e-4) * mb)
                for orow, lcol, off, go in (
                        (orow_t, lcol_t, offt, got),
                        (orow_b, lcol_b, offb, gob)):
                    for c in range(2):
                        d = _gather_hw(orow, lcol, off[b, c]) - go[b, c, :]
                        ad = jnp.abs(d)
                        l = jnp.where(ad < 1.0, 0.5 * d * d, ad - 0.5)
                        osum = osum + jnp.sum(l * mb)

            acc[4] = att + pull + osum / (num_tot + 1e-4)

        @pl.when(i == nsteps - 1)
        def _final():
            total = -acc[0] / acc[1] - acc[2] / acc[3] + acc[4]
            out[...] = jnp.broadcast_to(total, (1, 1))

    return body


def kernel(tl_heat, br_heat, tl_tag, br_tag, tl_off, br_off,
           att0, att1, att2, gt_tl_heat, gt_br_heat, gt_mask,
           gt_tl_off, gt_br_off, gt_tl_ind, gt_br_ind,
           gt_tl_valid, gt_br_valid, gt_att0, gt_att1, gt_att2):
    B, C, H, W = tl_heat.shape
    K = gt_mask.shape[1]
    R = B * H
    ROWS = 16
    nsteps = R // ROWS

    big = [jnp.transpose(a, (0, 2, 3, 1)).reshape(R, W, C) for a in
           (tl_heat, br_heat, gt_tl_heat, gt_br_heat,
            gt_tl_valid, gt_br_valid)]
    a0 = att0.reshape(B, *att0.shape[2:])
    ga0 = gt_att0.reshape(B, *gt_att0.shape[2:])
    a1 = att1.reshape(B, *att1.shape[2:])
    ga1 = gt_att1.reshape(B, *gt_att1.shape[2:])
    a2 = att2.reshape(B, *att2.shape[2:])
    ga2 = gt_att2.reshape(B, *gt_att2.shape[2:])
    tag_tl = tl_tag.reshape(B, H, W)
    tag_br = br_tag.reshape(B, H, W)
    ind_tl = gt_tl_ind.astype(jnp.int32)
    ind_br = gt_br_ind.astype(jnp.int32)
    maskf = gt_mask.astype(jnp.float32)
    goff_tl = jnp.transpose(gt_tl_off, (0, 2, 1))
    goff_br = jnp.transpose(gt_br_off, (0, 2, 1))

    big_spec = pl.BlockSpec((ROWS, W, C), lambda i: (i, 0, 0))
    full = lambda shape: pl.BlockSpec(shape, lambda i: (0,) * len(shape))

    res = pl.pallas_call(
        _make_body(nsteps, B, K, H, W),
        grid=(nsteps,),
        in_specs=[big_spec] * 6 + [
            full(a0.shape), full(ga0.shape),
            full(a1.shape), full(ga1.shape),
            full(a2.shape), full(ga2.shape),
            full(tag_tl.shape), full(tag_br.shape),
            full(tl_off.shape), full(br_off.shape),
            full(ind_tl.shape), full(ind_br.shape),
            full(maskf.shape),
            full(goff_tl.shape), full(goff_br.shape),
        ],
        out_specs=pl.BlockSpec((1, 1), lambda i: (0, 0)),
        out_shape=jax.ShapeDtypeStruct((1, 1), jnp.float32),
        scratch_shapes=[pltpu.SMEM((5,), jnp.float32)],
        compiler_params=pltpu.CompilerParams(
            dimension_semantics=("arbitrary",)),
    )(*big, a0, ga0, a1, ga1, a2, ga2,
      tag_tl, tag_br, tl_off, br_off, ind_tl, ind_br, maskf,
      goff_tl, goff_br)
    return res.reshape(1)
